# baseline (speedup 1.0000x reference)
import functools

import jax
import jax.numpy as jnp
from jax import lax
from jax.experimental import pallas as pl
from jax.experimental.pallas import tpu as pltpu
from jax.experimental.pallas import tpu_sc as plsc

B = 16384
D = 64
L = 16
NC, NS = 2, 16
NW = NC * NS
RPW = B // NW
HB = RPW // 2
CH = 128
NBLK = HB // L

_mesh = plsc.VectorSubcoreMesh(core_axis_name="c", subcore_axis_name="s")


@functools.partial(
    pl.kernel,
    mesh=_mesh,
    compiler_params=pltpu.CompilerParams(needs_layout_passes=False),
    out_type=jax.ShapeDtypeStruct((B,), jnp.float32),
    scratch_types=[
        pltpu.VMEM((RPW,), jnp.int32),
        pltpu.VMEM((RPW,), jnp.int32),
        pltpu.VMEM((RPW,), jnp.int32),
        pltpu.VMEM((RPW,), jnp.int32),
        pltpu.VMEM((HB, 2 * D), jnp.float32),
        pltpu.VMEM((HB, 2 * D), jnp.float32),
        pltpu.VMEM((9 * L,), jnp.float32),
        pltpu.VMEM((4 * L * L,), jnp.float32),
        pltpu.VMEM((RPW,), jnp.float32),
        pltpu.SemaphoreType.DMA,
    ],
)
def _ncf_sc(uphys, iphys, upar, ipar, utab, itab, wb, out,
            uidx_v, iidx_v, upar_v, ipar_v, ubuf, ibuf, w_v, t_v, out_v, sem):
    wid = lax.axis_index("s") * NC + lax.axis_index("c")
    base = wid * RPW
    pltpu.sync_copy(uphys.at[pl.ds(base, RPW)], uidx_v)
    pltpu.sync_copy(iphys.at[pl.ds(base, RPW)], iidx_v)
    pltpu.sync_copy(upar.at[pl.ds(base, RPW)], upar_v)
    pltpu.sync_copy(ipar.at[pl.ds(base, RPW)], ipar_v)
    pltpu.sync_copy(wb, w_v)

    wu = [w_v[pl.ds(16 * j, 16)] for j in range(4)]
    wi = [w_v[pl.ds(D + 16 * j, 16)] for j in range(4)]
    bias = w_v[pl.ds(2 * D, 16)]
    lanes16 = lax.iota(jnp.int32, L) * L

    for h in range(2):
        handles = []
        for c in range(HB // CH):
            off = h * HB + c * CH
            handles.append(pltpu.async_copy(
                utab.at[uidx_v.at[pl.ds(off, CH)]],
                ubuf.at[pl.ds(c * CH, CH)], sem))
            handles.append(pltpu.async_copy(
                itab.at[iidx_v.at[pl.ds(off, CH)]],
                ibuf.at[pl.ds(c * CH, CH)], sem))
        for hd in handles:
            hd.wait()

        def blk(b2, carry):
            lr0 = b2 * L
            g0 = h * HB + lr0
            for r in range(L):
                ul = ubuf[lr0 + r, pl.ds(0, 16)] * wu[0]
                uh = ubuf[lr0 + r, pl.ds(D, 16)] * wu[0]
                il = ibuf[lr0 + r, pl.ds(0, 16)] * wi[0]
                ih = ibuf[lr0 + r, pl.ds(D, 16)] * wi[0]
                for j in range(1, 4):
                    ul = ul + ubuf[lr0 + r, pl.ds(16 * j, 16)] * wu[j]
                    uh = uh + ubuf[lr0 + r, pl.ds(D + 16 * j, 16)] * wu[j]
                    il = il + ibuf[lr0 + r, pl.ds(16 * j, 16)] * wi[j]
                    ih = ih + ibuf[lr0 + r, pl.ds(D + 16 * j, 16)] * wi[j]
                plsc.store_scatter(t_v, [lanes16 + r], ul)
                plsc.store_scatter(t_v, [lanes16 + (256 + r)], uh)
                plsc.store_scatter(t_v, [lanes16 + (512 + r)], il)
                plsc.store_scatter(t_v, [lanes16 + (768 + r)], ih)
            vul = t_v[pl.ds(0, 16)]
            vuh = t_v[pl.ds(256, 16)]
            vil = t_v[pl.ds(512, 16)]
            vih = t_v[pl.ds(768, 16)]
            for k in range(1, L):
                vul = vul + t_v[pl.ds(k * 16, 16)]
                vuh = vuh + t_v[pl.ds(256 + k * 16, 16)]
                vil = vil + t_v[pl.ds(512 + k * 16, 16)]
                vih = vih + t_v[pl.ds(768 + k * 16, 16)]
            up16 = upar_v[pl.ds(g0, 16)]
            ip16 = ipar_v[pl.ds(g0, 16)]
            res = (bias
                   + jnp.where(up16 > 0, vuh, vul)
                   + jnp.where(ip16 > 0, vih, vil))
            out_v[pl.ds(g0, 16)] = res
            return carry

        lax.fori_loop(0, NBLK, blk, 0)

    pltpu.sync_copy(out_v, out.at[pl.ds(base, RPW)])


def kernel(user_ids, item_ids, user_table, item_table, fc_w, fc_b):
    uphys = jnp.right_shift(user_ids, 1)
    iphys = jnp.right_shift(item_ids, 1)
    upar = jnp.bitwise_and(user_ids, 1)
    ipar = jnp.bitwise_and(item_ids, 1)
    ut2 = user_table.reshape(user_table.shape[0] // 2, 2 * D)
    it2 = item_table.reshape(item_table.shape[0] // 2, 2 * D)
    wb = jnp.concatenate([fc_w[:, 0], jnp.broadcast_to(fc_b, (L,))])
    out = _ncf_sc(uphys, iphys, upar, ipar, ut2, it2, wb)
    return out.reshape(B, 1)


# speedup vs baseline: 3.7889x; 3.7889x over previous
import functools

import jax
import jax.numpy as jnp
from jax import lax
from jax.experimental import pallas as pl
from jax.experimental.pallas import tpu as pltpu
from jax.experimental.pallas import tpu_sc as plsc

B = 16384
D = 64
V = 1000000
L = 16
NC, NS = 2, 16
NW = NC * NS
RPW = B // NW
CH = 128
CB = 8192

_mesh = plsc.VectorSubcoreMesh(core_axis_name="c", subcore_axis_name="s")


def _wsum_body(ut_ref, it_ref, wu_ref, wi_ref, vu_ref, vi_ref):
    vu_ref[0, :] = jnp.sum(ut_ref[...] * wu_ref[...], axis=0)
    vi_ref[0, :] = jnp.sum(it_ref[...] * wi_ref[...], axis=0)


_NCB = (V + CB - 1) // CB

_wsum = pl.pallas_call(
    _wsum_body,
    grid=(_NCB,),
    in_specs=[
        pl.BlockSpec((D, CB), lambda j: (0, j)),
        pl.BlockSpec((D, CB), lambda j: (0, j)),
        pl.BlockSpec((D, 1), lambda j: (0, 0)),
        pl.BlockSpec((D, 1), lambda j: (0, 0)),
    ],
    out_specs=[
        pl.BlockSpec((1, CB), lambda j: (0, j)),
        pl.BlockSpec((1, CB), lambda j: (0, j)),
    ],
    out_shape=[
        jax.ShapeDtypeStruct((1, V), jnp.float32),
        jax.ShapeDtypeStruct((1, V), jnp.float32),
    ],
)


@functools.partial(
    pl.kernel,
    mesh=_mesh,
    out_type=jax.ShapeDtypeStruct((B,), jnp.float32),
    scratch_types=[
        pltpu.VMEM((RPW,), jnp.int32),
        pltpu.VMEM((RPW,), jnp.int32),
        pltpu.VMEM((RPW,), jnp.float32),
        pltpu.VMEM((RPW,), jnp.float32),
        pltpu.VMEM((L,), jnp.float32),
        pltpu.VMEM((RPW,), jnp.float32),
        pltpu.SemaphoreType.DMA,
    ],
)
def _gather_sc(uids, iids, vu, vi, bvec, out,
               uidx_v, iidx_v, gu_v, gi_v, b_v, out_v, sem):
    wid = lax.axis_index("s") * NC + lax.axis_index("c")
    base = wid * RPW
    pltpu.sync_copy(uids.at[pl.ds(base, RPW)], uidx_v)
    pltpu.sync_copy(iids.at[pl.ds(base, RPW)], iidx_v)
    pltpu.sync_copy(bvec, b_v)
    handles = []
    for c in range(RPW // CH):
        handles.append(pltpu.async_copy(
            vu.at[uidx_v.at[pl.ds(c * CH, CH)]],
            gu_v.at[pl.ds(c * CH, CH)], sem))
        handles.append(pltpu.async_copy(
            vi.at[iidx_v.at[pl.ds(c * CH, CH)]],
            gi_v.at[pl.ds(c * CH, CH)], sem))
    for h in handles:
        h.wait()
    bias = b_v[...]
    for k in range(RPW // L):
        out_v[pl.ds(k * L, L)] = (gu_v[pl.ds(k * L, L)]
                                  + gi_v[pl.ds(k * L, L)] + bias)
    pltpu.sync_copy(out_v, out.at[pl.ds(base, RPW)])


def kernel(user_ids, item_ids, user_table, item_table, fc_w, fc_b):
    ut_t = user_table.T
    it_t = item_table.T
    wu = fc_w[:D]
    wi = fc_w[D:]
    vu, vi = _wsum(ut_t, it_t, wu, wi)
    bvec = jnp.broadcast_to(fc_b, (L,))
    out = _gather_sc(user_ids, item_ids, vu.reshape(V), vi.reshape(V), bvec)
    return out.reshape(B, 1)


# speedup vs baseline: 4.2236x; 1.1147x over previous
import functools

import jax
import jax.numpy as jnp
from jax import lax
from jax.experimental import pallas as pl
from jax.experimental.pallas import tpu as pltpu
from jax.experimental.pallas import tpu_sc as plsc

B = 16384
D = 64
V = 1000000
L = 16
NC, NS = 2, 16
NW = NC * NS
RPW = B // NW
CH = 128
CB = 16384

_mesh = plsc.VectorSubcoreMesh(core_axis_name="c", subcore_axis_name="s")


def _wsum_body(ut_ref, it_ref, wu_ref, wi_ref, vu_ref, vi_ref):
    vu_ref[0, :] = jnp.sum(ut_ref[...] * wu_ref[...], axis=0)
    vi_ref[0, :] = jnp.sum(it_ref[...] * wi_ref[...], axis=0)


_NCB = (V + CB - 1) // CB

_wsum = pl.pallas_call(
    _wsum_body,
    grid=(_NCB,),
    in_specs=[
        pl.BlockSpec((D, CB), lambda j: (0, j)),
        pl.BlockSpec((D, CB), lambda j: (0, j)),
        pl.BlockSpec((D, 1), lambda j: (0, 0)),
        pl.BlockSpec((D, 1), lambda j: (0, 0)),
    ],
    out_specs=[
        pl.BlockSpec((1, CB), lambda j: (0, j)),
        pl.BlockSpec((1, CB), lambda j: (0, j)),
    ],
    out_shape=[
        jax.ShapeDtypeStruct((1, V), jnp.float32),
        jax.ShapeDtypeStruct((1, V), jnp.float32),
    ],
)


@functools.partial(
    pl.kernel,
    mesh=_mesh,
    out_type=jax.ShapeDtypeStruct((B,), jnp.float32),
    scratch_types=[
        pltpu.VMEM((RPW,), jnp.int32),
        pltpu.VMEM((RPW,), jnp.int32),
        pltpu.VMEM((RPW,), jnp.float32),
        pltpu.VMEM((RPW,), jnp.float32),
        pltpu.VMEM((L,), jnp.float32),
        pltpu.VMEM((RPW,), jnp.float32),
        pltpu.SemaphoreType.DMA,
    ],
)
def _gather_sc(uids, iids, vu, vi, bvec, out,
               uidx_v, iidx_v, gu_v, gi_v, b_v, out_v, sem):
    wid = lax.axis_index("s") * NC + lax.axis_index("c")
    base = wid * RPW
    pltpu.sync_copy(uids.at[pl.ds(base, RPW)], uidx_v)
    pltpu.sync_copy(iids.at[pl.ds(base, RPW)], iidx_v)
    pltpu.sync_copy(bvec, b_v)
    handles = []
    for c in range(RPW // CH):
        handles.append(pltpu.async_copy(
            vu.at[uidx_v.at[pl.ds(c * CH, CH)]],
            gu_v.at[pl.ds(c * CH, CH)], sem))
        handles.append(pltpu.async_copy(
            vi.at[iidx_v.at[pl.ds(c * CH, CH)]],
            gi_v.at[pl.ds(c * CH, CH)], sem))
    for h in handles:
        h.wait()
    bias = b_v[...]
    for k in range(RPW // L):
        out_v[pl.ds(k * L, L)] = (gu_v[pl.ds(k * L, L)]
                                  + gi_v[pl.ds(k * L, L)] + bias)
    pltpu.sync_copy(out_v, out.at[pl.ds(base, RPW)])


def kernel(user_ids, item_ids, user_table, item_table, fc_w, fc_b):
    ut_t = user_table.T
    it_t = item_table.T
    wu = fc_w[:D]
    wi = fc_w[D:]
    vu, vi = _wsum(ut_t, it_t, wu, wi)
    bvec = jnp.broadcast_to(fc_b, (L,))
    out = _gather_sc(user_ids, item_ids, vu.reshape(V), vi.reshape(V), bvec)
    return out.reshape(B, 1)
